# baseline (speedup 1.0000x reference)
import functools

import jax
import jax.numpy as jnp
from jax import lax
from jax.experimental import pallas as pl
from jax.experimental.pallas import tpu as pltpu
from jax.experimental.pallas import tpu_sc as plsc

_H = 224
_W = 224
_HW = _W // 2
_F = 4
_DSUB = 128
_LANES = 16
_VJ = _DSUB // _LANES
_NWORKERS = 32
_SLABS = _F * _H
_PER_W = _SLABS // _NWORKERS


def _materialize(cw, rw, fw):
    mesh = plsc.VectorSubcoreMesh(core_axis_name="c", subcore_axis_name="s")

    @functools.partial(
        pl.kernel,
        mesh=mesh,
        out_type=jax.ShapeDtypeStruct((_F, _H, _W, 3 * _DSUB), jnp.float32),
        scratch_types=[
            pltpu.VMEM((_HW, _DSUB), jnp.float32),
            pltpu.VMEM((_HW, _DSUB), jnp.float32),
            pltpu.VMEM((_H, _DSUB), jnp.float32),
            pltpu.VMEM((_F, _DSUB), jnp.float32),
            pltpu.VMEM_SHARED((_W, _DSUB), jnp.float32),
            pltpu.VMEM_SHARED((_F, _W, _DSUB), jnp.float32),
            pltpu.SemaphoreType.DMA,
            pltpu.SemaphoreType.DMA,
            pltpu.SemaphoreType.DMA,
        ],
    )
    def kern(cw_hbm, rw_hbm, fw_hbm, x_hbm, cbuf_a, cbuf_b, cstage, fstage,
             rshared, fshared, sem_a, sem_b, sem_rf):
        cc = lax.axis_index("c")
        ss = lax.axis_index("s")
        wid = ss * 2 + cc
        per_k = _H // _PER_W
        kk = wid // per_k
        ibase = (wid % per_k) * _PER_W

        stage = [
            pltpu.make_async_copy(cw_hbm.at[pl.ds(0, _H)], cstage, sem_a),
            pltpu.make_async_copy(fw_hbm.at[pl.ds(0, _F)], fstage, sem_a),
        ]
        for c in stage:
            c.start()

        @pl.when(ss == 0)
        def _():
            pltpu.sync_copy(rw_hbm.at[pl.ds(0, _W)], rshared)

        for c in stage:
            c.wait()

        def fill(buf, vecs, rows):
            def row(j, carry):
                for m in range(_VJ):
                    buf[j, pl.ds(m * _LANES, _LANES)] = vecs[m]
                return carry

            lax.fori_loop(0, rows, row, 0)

        @pl.when(ss % 4 == 0)
        def _():
            fill(cbuf_a,
                 [fstage[kk, pl.ds(m * _LANES, _LANES)] for m in range(_VJ)],
                 _HW)
            for jlo in (0, _HW):
                pltpu.sync_copy(cbuf_a, fshared.at[kk, pl.ds(jlo, _HW)])

        plsc.subcore_barrier()

        def start2(buf, ii, dlo, sem):
            for jlo in (0, _HW):
                pltpu.make_async_copy(
                    buf, x_hbm.at[kk, ii, pl.ds(jlo, _HW), pl.ds(dlo, _DSUB)],
                    sem).start()

        def wait2(buf, dlo, sem):
            for jlo in (0, _HW):
                pltpu.make_async_copy(
                    buf, x_hbm.at[kk, ibase, pl.ds(jlo, _HW),
                                  pl.ds(dlo, _DSUB)], sem).wait()

        def step(p, carry):
            ii = ibase + p

            pltpu.make_async_copy(
                rshared, x_hbm.at[kk, ii, :, pl.ds(_DSUB, _DSUB)],
                sem_rf).start()
            pltpu.make_async_copy(
                fshared.at[kk], x_hbm.at[kk, ii, :, pl.ds(2 * _DSUB, _DSUB)],
                sem_rf).start()

            @pl.when(p % 2 == 0)
            def _():
                @pl.when(p >= 2)
                def _():
                    wait2(cbuf_a, 0, sem_a)

                fill(cbuf_a, [cstage[ii, pl.ds(m * _LANES, _LANES)]
                              for m in range(_VJ)], _HW)
                start2(cbuf_a, ii, 0, sem_a)

            @pl.when(p % 2 == 1)
            def _():
                @pl.when(p >= 3)
                def _():
                    wait2(cbuf_b, 0, sem_b)

                fill(cbuf_b, [cstage[ii, pl.ds(m * _LANES, _LANES)]
                              for m in range(_VJ)], _HW)
                start2(cbuf_b, ii, 0, sem_b)

            return carry

        lax.fori_loop(0, _PER_W, step, 0)

        wait2(cbuf_a, 0, sem_a)
        wait2(cbuf_b, 0, sem_b)

        def drain(p, carry):
            pltpu.make_async_copy(
                rshared, x_hbm.at[kk, ibase, :, pl.ds(_DSUB, _DSUB)],
                sem_rf).wait()
            pltpu.make_async_copy(
                fshared.at[kk], x_hbm.at[kk, ibase, :, pl.ds(2 * _DSUB, _DSUB)],
                sem_rf).wait()
            return carry

        lax.fori_loop(0, _PER_W, drain, 0)
        plsc.subcore_barrier()

    return kern(cw, rw, fw)


def kernel(patch, num_views, row_weight, col_weight, frame_weight):
    x = _materialize(col_weight, row_weight, frame_weight)
    return jnp.transpose(x, (0, 3, 1, 2))


# speedup vs baseline: 1.1362x; 1.1362x over previous
import functools

import jax
import jax.numpy as jnp
from jax import lax
from jax.experimental import pallas as pl
from jax.experimental.pallas import tpu as pltpu
from jax.experimental.pallas import tpu_sc as plsc

_H = 224
_W = 224
_HW = _W // 2
_F = 4
_DSUB = 128
_LANES = 16
_VJ = _DSUB // _LANES
_NWORKERS = 32
_SLABS = _F * _H
_PER_W = _SLABS // _NWORKERS


def _materialize(cw, rw, fw):
    mesh = plsc.VectorSubcoreMesh(core_axis_name="c", subcore_axis_name="s")

    @functools.partial(
        pl.kernel,
        mesh=mesh,
        out_type=jax.ShapeDtypeStruct((_F, _H, _W, 3 * _DSUB), jnp.float32),
        scratch_types=[
            pltpu.VMEM((_HW, _DSUB), jnp.float32),
            pltpu.VMEM((_HW, _DSUB), jnp.float32),
            pltpu.VMEM((_W, _DSUB), jnp.float32),
            pltpu.VMEM((_W, _DSUB), jnp.float32),
            pltpu.VMEM((_H, _DSUB), jnp.float32),
            pltpu.VMEM((_F, _DSUB), jnp.float32),
            pltpu.SemaphoreType.DMA,
            pltpu.SemaphoreType.DMA,
            pltpu.SemaphoreType.DMA,
        ],
    )
    def kern(cw_hbm, rw_hbm, fw_hbm, x_hbm, cbuf_a, cbuf_b, rbuf, fbuf,
             cstage, fstage, sem_a, sem_b, sem_rf):
        wid = lax.axis_index("s") * 2 + lax.axis_index("c")
        per_k = _H // _PER_W
        kk = wid // per_k
        ibase = (wid % per_k) * _PER_W

        stage = [
            pltpu.make_async_copy(rw_hbm.at[pl.ds(0, _W)], rbuf, sem_a),
            pltpu.make_async_copy(cw_hbm.at[pl.ds(0, _H)], cstage, sem_a),
            pltpu.make_async_copy(fw_hbm.at[pl.ds(0, _F)], fstage, sem_a),
        ]
        for c in stage:
            c.start()
        for c in stage:
            c.wait()

        def fill(buf, vecs):
            def row(j, carry):
                for m in range(_VJ):
                    buf[j, pl.ds(m * _LANES, _LANES)] = vecs[m]
                return carry

            lax.fori_loop(0, _HW, row, 0)

        def start2(buf, ii, dlo, sem):
            for jlo in (0, _HW):
                pltpu.make_async_copy(
                    buf, x_hbm.at[kk, ii, pl.ds(jlo, _HW), pl.ds(dlo, _DSUB)],
                    sem).start()

        def wait2(buf, dlo, sem):
            for jlo in (0, _HW):
                pltpu.make_async_copy(
                    buf, x_hbm.at[kk, ibase, pl.ds(jlo, _HW),
                                  pl.ds(dlo, _DSUB)], sem).wait()

        def ffill(j, carry):
            for m in range(_VJ):
                fbuf[j, pl.ds(m * _LANES, _LANES)] = fvecs[m]
            return carry

        fvecs = [fstage[kk, pl.ds(m * _LANES, _LANES)] for m in range(_VJ)]
        lax.fori_loop(0, _W, ffill, 0)

        def step(p, carry):
            ii = ibase + p

            pltpu.make_async_copy(
                rbuf, x_hbm.at[kk, ii, :, pl.ds(_DSUB, _DSUB)], sem_rf).start()
            pltpu.make_async_copy(
                fbuf, x_hbm.at[kk, ii, :, pl.ds(2 * _DSUB, _DSUB)],
                sem_rf).start()

            @pl.when(p % 2 == 0)
            def _():
                @pl.when(p >= 2)
                def _():
                    wait2(cbuf_a, 0, sem_a)

                fill(cbuf_a, [cstage[ii, pl.ds(m * _LANES, _LANES)]
                              for m in range(_VJ)])
                start2(cbuf_a, ii, 0, sem_a)

            @pl.when(p % 2 == 1)
            def _():
                @pl.when(p >= 3)
                def _():
                    wait2(cbuf_b, 0, sem_b)

                fill(cbuf_b, [cstage[ii, pl.ds(m * _LANES, _LANES)]
                              for m in range(_VJ)])
                start2(cbuf_b, ii, 0, sem_b)

            return carry

        lax.fori_loop(0, _PER_W, step, 0)

        wait2(cbuf_a, 0, sem_a)
        wait2(cbuf_b, 0, sem_b)

        def drain(p, carry):
            pltpu.make_async_copy(
                rbuf, x_hbm.at[kk, ibase, :, pl.ds(_DSUB, _DSUB)],
                sem_rf).wait()
            pltpu.make_async_copy(
                fbuf, x_hbm.at[kk, ibase, :, pl.ds(2 * _DSUB, _DSUB)],
                sem_rf).wait()
            return carry

        lax.fori_loop(0, _PER_W, drain, 0)

    return kern(cw, rw, fw)


def kernel(patch, num_views, row_weight, col_weight, frame_weight):
    x = _materialize(col_weight, row_weight, frame_weight)
    return jnp.transpose(x, (0, 3, 1, 2))
